# Initial kernel scaffold; baseline (speedup 1.0000x reference)
import jax
import jax.numpy as jnp
from jax.experimental import pallas as pl


def kernel(x, edge_index, W_self1, W_neigh1, b1, W_self2, W_neigh2, b2, W_self3, W_neigh3, b3):
    raise NotImplementedError("write your pallas kernel here")


# speedup vs baseline: 2.4373x; 2.4373x over previous
import functools

import jax
import jax.numpy as jnp
from jax import lax
from jax.experimental import pallas as pl
from jax.experimental.pallas import tpu as pltpu
from jax.experimental.pallas import tpu_sc as plsc

_N = 10000
_E = 160000
_D = 256
_H = 256
_C = 64

_NP = 10240
_NTILES = 32
_BPT = 40
_EB = _NTILES * _BPT * 128
_BN = 1024
_NB = _NP // _BN

_ROWS_PER_TILE = _NP // 16
_RPT_STEPS = _ROWS_PER_TILE // 128


def _make_sc_segsum(nchunks, w, with_deg):
  mesh = plsc.VectorSubcoreMesh(core_axis_name="c", subcore_axis_name="s")
  p_type = jax.ShapeDtypeStruct((2, nchunks, _NP, w), jnp.float32)
  out_type = [p_type] if with_deg else p_type
  scratch = [
      pltpu.VMEM((_BPT, 128), jnp.int32),
      pltpu.VMEM((_BPT, 128), jnp.int32),
      pltpu.VMEM((128, w), jnp.float32),
      pltpu.VMEM((128, w), jnp.float32),
      pltpu.VMEM_SHARED((_NP, w), jnp.float32),
      pltpu.SemaphoreType.DMA,
  ]
  if with_deg:
    out_type.append(jax.ShapeDtypeStruct((2, _NP), jnp.float32))
    scratch += [
        pltpu.VMEM((128,), jnp.float32),
        pltpu.VMEM_SHARED((_NP,), jnp.float32),
    ]

  def body(y_hbm, src_hbm, dst_hbm, *rest):
    if with_deg:
      (out_p, out_d, src_v, dst_v, gbuf, zbuf, acc, sem, ones_v, dacc) = rest
    else:
      (out_p, src_v, dst_v, gbuf, zbuf, acc, sem) = rest
      out_d = ones_v = dacc = None

    cid = lax.axis_index("c")
    sid = lax.axis_index("s")
    tid = cid * 16 + sid

    def zrow(r, _):
      def zcol(c, __):
        zbuf[r, pl.ds(c * 16, 16)] = jnp.zeros((16,), jnp.float32)
        return 0
      return lax.fori_loop(0, w // 16, zcol, 0)
    lax.fori_loop(0, 128, zrow, 0)
    if with_deg:
      def orow(c, _):
        ones_v[pl.ds(c * 16, 16)] = jnp.ones((16,), jnp.float32)
        return 0
      lax.fori_loop(0, 8, orow, 0)

    pltpu.sync_copy(dst_hbm.at[pl.ds(tid * _BPT, _BPT)], dst_v)

    for ch in range(nchunks):
      pltpu.sync_copy(src_hbm.at[ch].at[pl.ds(tid * _BPT, _BPT)], src_v)

      for k in range(_RPT_STEPS):
        rows = pl.ds(sid * _ROWS_PER_TILE + k * 128, 128)
        pltpu.sync_copy(zbuf, acc.at[rows])
        if with_deg and ch == 0:
          pltpu.sync_copy(zbuf.at[0], dacc.at[rows])
      plsc.subcore_barrier()

      def batch(j, _):
        pltpu.async_copy(y_hbm.at[src_v.at[j]], gbuf, sem).wait()
        pltpu.sync_copy(gbuf, acc.at[dst_v.at[j]], add=True)
        if with_deg and ch == 0:
          pltpu.sync_copy(ones_v, dacc.at[dst_v.at[j]], add=True)
        return 0
      lax.fori_loop(0, _BPT, batch, 0)
      plsc.subcore_barrier()

      for k in range(_RPT_STEPS):
        rows = pl.ds(sid * _ROWS_PER_TILE + k * 128, 128)
        pltpu.sync_copy(acc.at[rows], out_p.at[cid, ch].at[rows])
        if with_deg and ch == 0:
          pltpu.sync_copy(dacc.at[rows], out_d.at[cid].at[rows])
      plsc.subcore_barrier()

  return pl.kernel(body, mesh=mesh, out_type=out_type, scratch_types=scratch)


def _tc_in_body(x_ref, ws_ref, wn_ref, b_ref, s_ref, y_ref):
  xb = x_ref[...]
  s_ref[...] = (
      jnp.dot(xb, ws_ref[...], preferred_element_type=jnp.float32) + b_ref[...]
  )
  y = jnp.dot(xb, wn_ref[...], preferred_element_type=jnp.float32)
  y_ref[0] = y[:, :128]
  y_ref[1] = y[:, 128:]


def _tc_input_layer(xp, ws, wn, b):
  return pl.pallas_call(
      _tc_in_body,
      grid=(_NB,),
      in_specs=[
          pl.BlockSpec((_BN, _D), lambda i: (i, 0)),
          pl.BlockSpec((_D, _H), lambda i: (0, 0)),
          pl.BlockSpec((_D, _H), lambda i: (0, 0)),
          pl.BlockSpec((_H,), lambda i: (0,)),
      ],
      out_specs=[
          pl.BlockSpec((_BN, _H), lambda i: (i, 0)),
          pl.BlockSpec((2, _BN, 128), lambda i: (0, i, 0)),
      ],
      out_shape=[
          jax.ShapeDtypeStruct((_NP, _H), jnp.float32),
          jax.ShapeDtypeStruct((2, _NP, 128), jnp.float32),
      ],
  )(xp, ws, wn, b)


def _make_tc_mid(dout, dy, out_chunks):
  wout = dy // out_chunks

  def body(s_ref, p_ref, inv_ref, ws_ref, wn_ref, b_ref, s2_ref, y2_ref):
    inv = inv_ref[...]
    a0 = (p_ref[0, 0] + p_ref[1, 0]) * inv
    a1 = (p_ref[0, 1] + p_ref[1, 1]) * inv
    sb = s_ref[...]
    h0 = jnp.maximum(sb[:, :128] + a0, 0.0)
    h1 = jnp.maximum(sb[:, 128:] + a1, 0.0)
    ws = ws_ref[...]
    wn = wn_ref[...]
    s2_ref[...] = (
        jnp.dot(h0, ws[:128], preferred_element_type=jnp.float32)
        + jnp.dot(h1, ws[128:], preferred_element_type=jnp.float32)
        + b_ref[...]
    )
    y2 = (
        jnp.dot(h0, wn[:128], preferred_element_type=jnp.float32)
        + jnp.dot(h1, wn[128:], preferred_element_type=jnp.float32)
    )
    for c in range(out_chunks):
      y2_ref[c] = y2[:, c * wout:(c + 1) * wout]

  def run(s_prev, partials, invmat, ws, wn, b):
    return pl.pallas_call(
        body,
        grid=(_NB,),
        in_specs=[
            pl.BlockSpec((_BN, _H), lambda i: (i, 0)),
            pl.BlockSpec((2, 2, _BN, 128), lambda i: (0, 0, i, 0)),
            pl.BlockSpec((_BN, 128), lambda i: (i, 0)),
            pl.BlockSpec((_H, dout), lambda i: (0, 0)),
            pl.BlockSpec((_H, dy), lambda i: (0, 0)),
            pl.BlockSpec((dout,), lambda i: (0,)),
        ],
        out_specs=[
            pl.BlockSpec((_BN, dout), lambda i: (i, 0)),
            pl.BlockSpec((out_chunks, _BN, wout), lambda i: (0, i, 0)),
        ],
        out_shape=[
            jax.ShapeDtypeStruct((_NP, dout), jnp.float32),
            jax.ShapeDtypeStruct((out_chunks, _NP, wout), jnp.float32),
        ],
    )(s_prev, partials, invmat, ws, wn, b)

  return run


def _tc_out_body(s_ref, p_ref, inv_ref, o_ref):
  o_ref[...] = s_ref[...] + (
      p_ref[0, 0, :, :_C] + p_ref[1, 0, :, :_C]) * inv_ref[...]


def _tc_output_layer(s3, p3, inv64):
  return pl.pallas_call(
      _tc_out_body,
      grid=(_NB,),
      in_specs=[
          pl.BlockSpec((_BN, _C), lambda i: (i, 0)),
          pl.BlockSpec((2, 1, _BN, 128), lambda i: (0, 0, i, 0)),
          pl.BlockSpec((_BN, _C), lambda i: (i, 0)),
      ],
      out_specs=pl.BlockSpec((_BN, _C), lambda i: (i, 0)),
      out_shape=jax.ShapeDtypeStruct((_NP, _C), jnp.float32),
  )(s3, p3, inv64)


_sc_layer12 = _make_sc_segsum(nchunks=2, w=128, with_deg=True)
_sc_layer2 = _make_sc_segsum(nchunks=2, w=128, with_deg=False)
_sc_layer3 = _make_sc_segsum(nchunks=1, w=128, with_deg=False)
_tc_mid_h = _make_tc_mid(dout=_H, dy=_H, out_chunks=2)
_tc_mid_c = _make_tc_mid(dout=_C, dy=128, out_chunks=1)


@jax.jit
def kernel(x, edge_index, W_self1, W_neigh1, b1, W_self2, W_neigh2, b2,
           W_self3, W_neigh3, b3):
  xp = jnp.zeros((_NP, _D), jnp.float32).at[:_N].set(x)

  src = edge_index[0].astype(jnp.int32)
  dst = edge_index[1].astype(jnp.int32)
  pad = _EB - _E
  srcp = jnp.concatenate([src, jnp.zeros((pad,), jnp.int32)]).reshape(
      _NTILES * _BPT, 128)
  dstp = jnp.concatenate([dst, jnp.full((pad,), _N, jnp.int32)]).reshape(
      _NTILES * _BPT, 128)
  src2 = jnp.stack([srcp, srcp + _NP])
  src1 = srcp[None]

  s1, y1 = _tc_input_layer(xp, W_self1, W_neigh1, b1)
  p1, dgp = _sc_layer12(y1.reshape(2 * _NP, 128), src2, dstp)
  invdeg = 1.0 / jnp.maximum(dgp[0] + dgp[1], 1.0)
  inv128 = jnp.broadcast_to(invdeg[:, None], (_NP, 128))
  inv64 = inv128[:, :_C]

  s2, y2 = _tc_mid_h(s1, p1, inv128, W_self2, W_neigh2, b2)
  p2 = _sc_layer2(y2.reshape(2 * _NP, 128), src2, dstp)

  wn3p = jnp.zeros((_H, 128), jnp.float32).at[:, :_C].set(W_neigh3)
  s3, y3 = _tc_mid_c(s2, p2, inv128, W_self3, wn3p, b3)
  p3 = _sc_layer3(y3.reshape(_NP, 128), src1, dstp)

  out = _tc_output_layer(s3, p3, inv64)
  return out[:_N]


# speedup vs baseline: 2.7309x; 1.1205x over previous
import functools

import jax
import jax.numpy as jnp
from jax import lax
from jax.experimental import pallas as pl
from jax.experimental.pallas import tpu as pltpu
from jax.experimental.pallas import tpu_sc as plsc

_N = 10000
_E = 160000
_D = 256
_H = 256
_C = 64

_NP = 10240
_NTILES = 32
_BPT = 40
_EB = _NTILES * _BPT * 128
_BN = 1024
_NB = _NP // _BN

_ROWS_PER_TILE = _NP // 16
_RPT_STEPS = _ROWS_PER_TILE // 128


def _make_sc_segsum(nchunks, w, with_deg):
  nbuf = 2
  mesh = plsc.VectorSubcoreMesh(core_axis_name="c", subcore_axis_name="s")
  p_type = jax.ShapeDtypeStruct((2, nchunks, _NP, w), jnp.float32)
  out_type = [p_type] if with_deg else p_type
  scratch = [
      pltpu.VMEM((_BPT, 128), jnp.int32),
      pltpu.VMEM((_BPT, 128), jnp.int32),
  ] + [pltpu.VMEM((128, w), jnp.float32) for _ in range(nbuf)] + [
      pltpu.VMEM_SHARED((_NP, w), jnp.float32),
  ] + [pltpu.SemaphoreType.DMA for _ in range(nbuf)]
  if with_deg:
    out_type.append(jax.ShapeDtypeStruct((2, _NP), jnp.float32))
    scratch += [
        pltpu.VMEM((128,), jnp.float32),
        pltpu.VMEM_SHARED((_NP,), jnp.float32),
    ]

  def body(y_hbm, src_hbm, dst_hbm, *rest):
    if with_deg:
      out_p, out_d, src_v, dst_v = rest[:4]
      gbufs = rest[4:4 + nbuf]
      acc = rest[4 + nbuf]
      sems = rest[5 + nbuf:5 + 2 * nbuf]
      ones_v, dacc = rest[5 + 2 * nbuf:]
    else:
      out_p, src_v, dst_v = rest[:3]
      gbufs = rest[3:3 + nbuf]
      acc = rest[3 + nbuf]
      sems = rest[4 + nbuf:4 + 2 * nbuf]
      out_d = ones_v = dacc = None

    cid = lax.axis_index("c")
    sid = lax.axis_index("s")
    tid = cid * 16 + sid

    if with_deg:
      def orow(c, _):
        ones_v[pl.ds(c * 16, 16)] = jnp.ones((16,), jnp.float32)
        return 0
      lax.fori_loop(0, 8, orow, 0)

    pltpu.sync_copy(dst_hbm.at[pl.ds(tid * _BPT, _BPT)], dst_v)

    for ch in range(nchunks):
      pltpu.sync_copy(src_hbm.at[ch].at[pl.ds(tid * _BPT, _BPT)], src_v)

      def zrow(r, _):
        def zcol(c, __):
          gbufs[0][r, pl.ds(c * 16, 16)] = jnp.zeros((16,), jnp.float32)
          return 0
        return lax.fori_loop(0, w // 16, zcol, 0)
      lax.fori_loop(0, 128, zrow, 0)
      for k in range(_RPT_STEPS):
        rows = pl.ds(sid * _ROWS_PER_TILE + k * 128, 128)
        pltpu.sync_copy(gbufs[0], acc.at[rows])
        if with_deg and ch == 0:
          pltpu.sync_copy(gbufs[0].at[0], dacc.at[rows])
      plsc.subcore_barrier()

      for b in range(nbuf):
        pltpu.async_copy(y_hbm.at[src_v.at[b]], gbufs[b], sems[b])

      def group(g, _):
        for b in range(nbuf):
          j = g * nbuf + b
          pltpu.make_async_copy(
              y_hbm.at[pl.ds(0, 128)], gbufs[b], sems[b]).wait()
          pltpu.sync_copy(gbufs[b], acc.at[dst_v.at[j]], add=True)
          if with_deg and ch == 0:
            pltpu.sync_copy(ones_v, dacc.at[dst_v.at[j]], add=True)
          pltpu.async_copy(y_hbm.at[src_v.at[j + nbuf]], gbufs[b], sems[b])
        return 0
      lax.fori_loop(0, _BPT // nbuf - 1, group, 0)

      for b in range(nbuf):
        j = _BPT - nbuf + b
        pltpu.make_async_copy(
            y_hbm.at[pl.ds(0, 128)], gbufs[b], sems[b]).wait()
        pltpu.sync_copy(gbufs[b], acc.at[dst_v.at[j]], add=True)
        if with_deg and ch == 0:
          pltpu.sync_copy(ones_v, dacc.at[dst_v.at[j]], add=True)
      plsc.subcore_barrier()

      for k in range(_RPT_STEPS):
        rows = pl.ds(sid * _ROWS_PER_TILE + k * 128, 128)
        pltpu.sync_copy(acc.at[rows], out_p.at[cid, ch].at[rows])
        if with_deg and ch == 0:
          pltpu.sync_copy(dacc.at[rows], out_d.at[cid].at[rows])
      plsc.subcore_barrier()

  return pl.kernel(body, mesh=mesh, out_type=out_type, scratch_types=scratch)


def _tc_in_body(x_ref, ws_ref, wn_ref, b_ref, s_ref, y_ref):
  xb = x_ref[...]
  s_ref[...] = (
      jnp.dot(xb, ws_ref[...], preferred_element_type=jnp.float32) + b_ref[...]
  )
  y = jnp.dot(xb, wn_ref[...], preferred_element_type=jnp.float32)
  y_ref[0] = y[:, :128]
  y_ref[1] = y[:, 128:]


def _tc_input_layer(xp, ws, wn, b):
  return pl.pallas_call(
      _tc_in_body,
      grid=(_NB,),
      in_specs=[
          pl.BlockSpec((_BN, _D), lambda i: (i, 0)),
          pl.BlockSpec((_D, _H), lambda i: (0, 0)),
          pl.BlockSpec((_D, _H), lambda i: (0, 0)),
          pl.BlockSpec((_H,), lambda i: (0,)),
      ],
      out_specs=[
          pl.BlockSpec((_BN, _H), lambda i: (i, 0)),
          pl.BlockSpec((2, _BN, 128), lambda i: (0, i, 0)),
      ],
      out_shape=[
          jax.ShapeDtypeStruct((_NP, _H), jnp.float32),
          jax.ShapeDtypeStruct((2, _NP, 128), jnp.float32),
      ],
  )(xp, ws, wn, b)


def _make_tc_mid(dout, dy, out_chunks):
  wout = dy // out_chunks

  def body(s_ref, p_ref, inv_ref, ws_ref, wn_ref, b_ref, s2_ref, y2_ref):
    inv = inv_ref[...]
    a0 = (p_ref[0, 0] + p_ref[1, 0]) * inv
    a1 = (p_ref[0, 1] + p_ref[1, 1]) * inv
    sb = s_ref[...]
    h0 = jnp.maximum(sb[:, :128] + a0, 0.0)
    h1 = jnp.maximum(sb[:, 128:] + a1, 0.0)
    ws = ws_ref[...]
    wn = wn_ref[...]
    s2_ref[...] = (
        jnp.dot(h0, ws[:128], preferred_element_type=jnp.float32)
        + jnp.dot(h1, ws[128:], preferred_element_type=jnp.float32)
        + b_ref[...]
    )
    y2 = (
        jnp.dot(h0, wn[:128], preferred_element_type=jnp.float32)
        + jnp.dot(h1, wn[128:], preferred_element_type=jnp.float32)
    )
    for c in range(out_chunks):
      y2_ref[c] = y2[:, c * wout:(c + 1) * wout]

  def run(s_prev, partials, invmat, ws, wn, b):
    return pl.pallas_call(
        body,
        grid=(_NB,),
        in_specs=[
            pl.BlockSpec((_BN, _H), lambda i: (i, 0)),
            pl.BlockSpec((2, 2, _BN, 128), lambda i: (0, 0, i, 0)),
            pl.BlockSpec((_BN, 128), lambda i: (i, 0)),
            pl.BlockSpec((_H, dout), lambda i: (0, 0)),
            pl.BlockSpec((_H, dy), lambda i: (0, 0)),
            pl.BlockSpec((dout,), lambda i: (0,)),
        ],
        out_specs=[
            pl.BlockSpec((_BN, dout), lambda i: (i, 0)),
            pl.BlockSpec((out_chunks, _BN, wout), lambda i: (0, i, 0)),
        ],
        out_shape=[
            jax.ShapeDtypeStruct((_NP, dout), jnp.float32),
            jax.ShapeDtypeStruct((out_chunks, _NP, wout), jnp.float32),
        ],
    )(s_prev, partials, invmat, ws, wn, b)

  return run


def _tc_out_body(s_ref, p_ref, inv_ref, o_ref):
  o_ref[...] = s_ref[...] + (
      p_ref[0, 0, :, :_C] + p_ref[1, 0, :, :_C]) * inv_ref[...]


def _tc_output_layer(s3, p3, inv64):
  return pl.pallas_call(
      _tc_out_body,
      grid=(_NB,),
      in_specs=[
          pl.BlockSpec((_BN, _C), lambda i: (i, 0)),
          pl.BlockSpec((2, 1, _BN, 128), lambda i: (0, 0, i, 0)),
          pl.BlockSpec((_BN, _C), lambda i: (i, 0)),
      ],
      out_specs=pl.BlockSpec((_BN, _C), lambda i: (i, 0)),
      out_shape=jax.ShapeDtypeStruct((_NP, _C), jnp.float32),
  )(s3, p3, inv64)


_sc_layer12 = _make_sc_segsum(nchunks=2, w=128, with_deg=True)
_sc_layer2 = _make_sc_segsum(nchunks=2, w=128, with_deg=False)
_sc_layer3 = _make_sc_segsum(nchunks=1, w=128, with_deg=False)
_tc_mid_h = _make_tc_mid(dout=_H, dy=_H, out_chunks=2)
_tc_mid_c = _make_tc_mid(dout=_C, dy=128, out_chunks=1)


@jax.jit
def kernel(x, edge_index, W_self1, W_neigh1, b1, W_self2, W_neigh2, b2,
           W_self3, W_neigh3, b3):
  xp = jnp.zeros((_NP, _D), jnp.float32).at[:_N].set(x)

  src = edge_index[0].astype(jnp.int32)
  dst = edge_index[1].astype(jnp.int32)
  pad = _EB - _E
  srcp = jnp.concatenate([src, jnp.zeros((pad,), jnp.int32)]).reshape(
      _NTILES * _BPT, 128)
  dstp = jnp.concatenate([dst, jnp.full((pad,), _N, jnp.int32)]).reshape(
      _NTILES * _BPT, 128)
  src2 = jnp.stack([srcp, srcp + _NP])
  src1 = srcp[None]

  s1, y1 = _tc_input_layer(xp, W_self1, W_neigh1, b1)
  p1, dgp = _sc_layer12(y1.reshape(2 * _NP, 128), src2, dstp)
  invdeg = 1.0 / jnp.maximum(dgp[0] + dgp[1], 1.0)
  inv128 = jnp.broadcast_to(invdeg[:, None], (_NP, 128))
  inv64 = inv128[:, :_C]

  s2, y2 = _tc_mid_h(s1, p1, inv128, W_self2, W_neigh2, b2)
  p2 = _sc_layer2(y2.reshape(2 * _NP, 128), src2, dstp)

  wn3p = jnp.zeros((_H, 128), jnp.float32).at[:, :_C].set(W_neigh3)
  s3, y3 = _tc_mid_c(s2, p2, inv128, W_self3, wn3p, b3)
  p3 = _sc_layer3(y3.reshape(_NP, 128), src1, dstp)

  out = _tc_output_layer(s3, p3, inv64)
  return out[:_N]
